# Initial kernel scaffold; baseline (speedup 1.0000x reference)
import jax
import jax.numpy as jnp
from jax.experimental import pallas as pl


def kernel(x, b_from_a, a_from_b, W, b):
    raise NotImplementedError("write your pallas kernel here")


# speedup vs baseline: 3.3992x; 3.3992x over previous
import functools

import jax
import jax.numpy as jnp
from jax import lax
from jax.experimental import pallas as pl
from jax.experimental.pallas import tpu as pltpu
from jax.experimental.pallas import tpu_sc as plsc

_N, _E, _DEG, _D = 10000, 320000, 32, 128
_NW = 32
_NPAD = 10240
_NODES_PER_W = _NPAD // _NW
_CHUNK_NODES = 4
_EC = _CHUNK_NODES * _DEG
_CHUNKS = _NODES_PER_W // _CHUNK_NODES
_LANES = 16
_DV = _D // _LANES

_MM_BLK = 1024


def _linear_body(x_ref, w_ref, b_ref, h_ref):
    h = lax.dot_general(x_ref[...], w_ref[...], (((1,), (1,)), ((), ())),
                        preferred_element_type=jnp.float32)
    h = h + b_ref[...]
    i = pl.program_id(0)

    @pl.when(i == 0)
    def _zero_row0():
        row = lax.broadcasted_iota(jnp.int32, (_MM_BLK, _D), 0)
        h_ref[...] = jnp.where(row == 0, 0.0, h)

    @pl.when(i != 0)
    def _plain():
        h_ref[...] = h


def _linear(x_pad, w, b2d):
    return pl.pallas_call(
        _linear_body,
        grid=(_NPAD // _MM_BLK,),
        in_specs=[
            pl.BlockSpec((_MM_BLK, _D), lambda i: (i, 0)),
            pl.BlockSpec((_D, _D), lambda i: (0, 0)),
            pl.BlockSpec((1, _D), lambda i: (0, 0)),
        ],
        out_specs=pl.BlockSpec((_MM_BLK, _D), lambda i: (i, 0)),
        out_shape=jax.ShapeDtypeStruct((_NPAD, _D), jnp.float32),
    )(x_pad, w, b2d)


@functools.partial(
    pl.kernel,
    out_type=jax.ShapeDtypeStruct((_NPAD, _D), jnp.float32),
    mesh=plsc.VectorSubcoreMesh(core_axis_name="c", subcore_axis_name="s"),
    scratch_types=[
        pltpu.VMEM_SHARED((_NPAD, _D), jnp.float32),
        pltpu.VMEM((_CHUNKS, _EC), jnp.int32),
        pltpu.VMEM((_CHUNKS, _EC), jnp.int32),
        pltpu.VMEM((_EC, _D), jnp.float32),
        pltpu.VMEM((_CHUNK_NODES, _D), jnp.float32),
        pltpu.VMEM((_CHUNK_NODES, _D), jnp.float32),
        pltpu.SemaphoreType.DMA,
    ],
)
def _sc_gin(h_hbm, afb_hbm, bfa_hbm, out_hbm,
            h_sh, afb_v, a2a_v, rows_v, hself_v, out_v, sem):
    c = lax.axis_index("c")
    s = lax.axis_index("s")
    wid = s * 2 + c

    rows_per_s = _NPAD // 16
    pltpu.sync_copy(h_hbm.at[pl.ds(s * rows_per_s, rows_per_s)],
                    h_sh.at[pl.ds(s * rows_per_s, rows_per_s)])
    plsc.subcore_barrier()

    base_node = wid * _NODES_PER_W
    pltpu.sync_copy(afb_hbm.at[pl.ds(wid * _CHUNKS, _CHUNKS)], afb_v)

    def chunk_body(i, carry):
        pltpu.async_copy(bfa_hbm.at[afb_v.at[i]], a2a_v.at[i], sem).wait()
        pltpu.async_copy(h_sh.at[a2a_v.at[i]], rows_v, sem).wait()
        nb = base_node + i * _CHUNK_NODES
        pltpu.sync_copy(h_sh.at[pl.ds(nb, _CHUNK_NODES)], hself_v)
        for j in range(_CHUNK_NODES):
            accs = tuple(hself_v[j, pl.ds(d * _LANES, _LANES)]
                         for d in range(_DV))

            def k_body(k, a, _j=j):
                return tuple(
                    a[d] + rows_v[_j * _DEG + k, pl.ds(d * _LANES, _LANES)]
                    for d in range(_DV))

            accs = lax.fori_loop(0, _DEG, k_body, accs)
            for d in range(_DV):
                out_v[j, pl.ds(d * _LANES, _LANES)] = jnp.maximum(
                    accs[d], 0.0)
        pltpu.sync_copy(out_v, out_hbm.at[pl.ds(nb, _CHUNK_NODES)])
        return carry

    lax.fori_loop(0, _CHUNKS, chunk_body, 0)


def kernel(x, b_from_a, a_from_b, W, b):
    x_pad = jnp.pad(x, ((0, _NPAD - _N), (0, 0)))
    afb = jnp.pad(a_from_b, ((0, _NPAD - _N), (0, 0)))
    afb = afb.reshape(_NPAD * _DEG // _EC, _EC)
    h = _linear(x_pad, W, b.reshape(1, _D))
    out = _sc_gin(h, afb, b_from_a)
    return out[:_N]


# speedup vs baseline: 5.1957x; 1.5285x over previous
import functools

import jax
import jax.numpy as jnp
from jax import lax
from jax.experimental import pallas as pl
from jax.experimental.pallas import tpu as pltpu
from jax.experimental.pallas import tpu_sc as plsc

_N, _E, _DEG, _D = 10000, 320000, 32, 128
_NW = 32
_NPAD = 10240
_NODES_PER_W = _NPAD // _NW
_CHUNK_NODES = 4
_EC = _CHUNK_NODES * _DEG
_CHUNKS = _NODES_PER_W // _CHUNK_NODES
_GROUPS = 2
_GC = _CHUNKS // _GROUPS
_LANES = 16
_DV = _D // _LANES

_MM_BLK = 1024


def _linear_body(x_ref, w_ref, b_ref, h_ref):
    h = lax.dot_general(x_ref[...], w_ref[...], (((1,), (1,)), ((), ())),
                        preferred_element_type=jnp.float32)
    h = h + b_ref[...]
    i = pl.program_id(0)

    @pl.when(i == 0)
    def _zero_row0():
        row = lax.broadcasted_iota(jnp.int32, (_MM_BLK, _D), 0)
        h_ref[...] = jnp.where(row == 0, 0.0, h)

    @pl.when(i != 0)
    def _plain():
        h_ref[...] = h


def _linear(x_pad, w, b2d):
    return pl.pallas_call(
        _linear_body,
        grid=(_NPAD // _MM_BLK,),
        in_specs=[
            pl.BlockSpec((_MM_BLK, _D), lambda i: (i, 0)),
            pl.BlockSpec((_D, _D), lambda i: (0, 0)),
            pl.BlockSpec((1, _D), lambda i: (0, 0)),
        ],
        out_specs=pl.BlockSpec((_MM_BLK, _D), lambda i: (i, 0)),
        out_shape=jax.ShapeDtypeStruct((_NPAD, _D), jnp.float32),
    )(x_pad, w, b2d)


@functools.partial(
    pl.kernel,
    out_type=jax.ShapeDtypeStruct((_NPAD, _D), jnp.float32),
    mesh=plsc.VectorSubcoreMesh(core_axis_name="c", subcore_axis_name="s"),
    scratch_types=[
        pltpu.VMEM_SHARED((_NPAD, _D), jnp.float32),
        pltpu.VMEM((_GC, _EC), jnp.int32),
        pltpu.VMEM((_GC, _EC), jnp.int32),
        pltpu.VMEM((2, _EC, _D), jnp.float32),
        pltpu.VMEM((_CHUNK_NODES, _D), jnp.float32),
        pltpu.VMEM((2, _CHUNK_NODES, _D), jnp.float32),
        pltpu.SemaphoreType.DMA,
        pltpu.SemaphoreType.DMA,
        pltpu.SemaphoreType.DMA,
    ],
)
def _sc_gin(h_hbm, afb_hbm, bfa_hbm, out_hbm,
            h_sh, afb_v, a2a_v, rows_v, hself_v, out_v,
            sem_idx, sem_rows, sem_out):
    c = lax.axis_index("c")
    s = lax.axis_index("s")
    wid = s * 2 + c

    rows_per_s = _NPAD // 16
    pltpu.sync_copy(h_hbm.at[pl.ds(s * rows_per_s, rows_per_s)],
                    h_sh.at[pl.ds(s * rows_per_s, rows_per_s)])

    plsc.subcore_barrier()
    base_node = wid * _NODES_PER_W

    def gather_rows(i, buf):
        pltpu.async_copy(h_sh.at[a2a_v.at[i]], rows_v.at[buf], sem_rows)

    def wait_rows(i, buf):
        pltpu.make_async_copy(h_sh.at[a2a_v.at[i]], rows_v.at[buf],
                              sem_rows).wait()

    def wait_out(nb, buf):
        pltpu.make_async_copy(out_v.at[buf],
                              out_hbm.at[pl.ds(nb, _CHUNK_NODES)],
                              sem_out).wait()

    for g in range(_GROUPS):
        gstart = g * _GC

        pltpu.sync_copy(afb_hbm.at[pl.ds(wid * _CHUNKS + gstart, _GC)],
                        afb_v)

        def fire_a2a(i, carry):
            pltpu.async_copy(bfa_hbm.at[afb_v.at[i]], a2a_v.at[i], sem_idx)
            return carry

        lax.fori_loop(0, _GC, fire_a2a, 0)

        def drain_a2a(i, carry):
            pltpu.make_async_copy(bfa_hbm.at[afb_v.at[i]], a2a_v.at[i],
                                  sem_idx).wait()
            return carry

        lax.fori_loop(0, _GC, drain_a2a, 0)

        gather_rows(0, 0)

        def chunk_body(i, carry):
            buf = lax.rem(i, 2)
            nb = base_node + (gstart + i) * _CHUNK_NODES
            wait_rows(i, buf)

            @pl.when(i + 1 < _GC)
            def _prefetch():
                gather_rows(i + 1, 1 - buf)

            @pl.when(i >= 2)
            def _drain_store():
                wait_out(nb - 2 * _CHUNK_NODES, buf)

            pltpu.sync_copy(h_sh.at[pl.ds(nb, _CHUNK_NODES)], hself_v)
            for j in range(_CHUNK_NODES):
                accs = tuple(hself_v[j, pl.ds(d * _LANES, _LANES)]
                             for d in range(_DV))

                def k_body(k, a, _j=j):
                    return tuple(
                        a[d] + rows_v[buf, _j * _DEG + k,
                                      pl.ds(d * _LANES, _LANES)]
                        for d in range(_DV))

                accs = lax.fori_loop(0, _DEG, k_body, accs, unroll=8)
                for d in range(_DV):
                    out_v[buf, j, pl.ds(d * _LANES, _LANES)] = jnp.maximum(
                        accs[d], 0.0)

            pltpu.async_copy(out_v.at[buf],
                             out_hbm.at[pl.ds(nb, _CHUNK_NODES)], sem_out)
            return carry

        lax.fori_loop(0, _GC, chunk_body, 0)

        gend = base_node + (gstart + _GC) * _CHUNK_NODES
        wait_out(gend - 2 * _CHUNK_NODES, 0)
        wait_out(gend - _CHUNK_NODES, 1)


def kernel(x, b_from_a, a_from_b, W, b):
    x_pad = jnp.pad(x, ((0, _NPAD - _N), (0, 0)))
    afb = jnp.pad(a_from_b, ((0, _NPAD - _N), (0, 0)))
    afb = afb.reshape(_NPAD * _DEG // _EC, _EC)
    h = _linear(x_pad, W, b.reshape(1, _D))
    out = _sc_gin(h, afb, b_from_a)
    return out[:_N]


# speedup vs baseline: 7.2009x; 1.3860x over previous
import functools

import jax
import jax.numpy as jnp
from jax import lax
from jax.experimental import pallas as pl
from jax.experimental.pallas import tpu as pltpu
from jax.experimental.pallas import tpu_sc as plsc

_N, _E, _DEG, _D = 10000, 320000, 32, 128
_NW = 32
_CHUNK_NODES = 4
_EC = _CHUNK_NODES * _DEG
_CHUNKS = _N // _CHUNK_NODES
_CPW = _CHUNKS // _NW
_REM = _CHUNKS - _CPW * _NW
_H1 = 40
_H2 = 39
_LANES = 16
_DV = _D // _LANES

_MM_BLK = 1000


def _linear_body(x_ref, w_ref, b_ref, h_ref):
    h = lax.dot_general(x_ref[...], w_ref[...], (((1,), (1,)), ((), ())),
                        preferred_element_type=jnp.float32)
    h = h + b_ref[...]
    i = pl.program_id(0)

    @pl.when(i == 0)
    def _zero_row0():
        row = lax.broadcasted_iota(jnp.int32, (_MM_BLK, _D), 0)
        h_ref[...] = jnp.where(row == 0, 0.0, h)

    @pl.when(i != 0)
    def _plain():
        h_ref[...] = h


def _linear(x, w, b2d):
    return pl.pallas_call(
        _linear_body,
        grid=(_N // _MM_BLK,),
        in_specs=[
            pl.BlockSpec((_MM_BLK, _D), lambda i: (i, 0)),
            pl.BlockSpec((_D, _D), lambda i: (0, 0)),
            pl.BlockSpec((1, _D), lambda i: (0, 0)),
        ],
        out_specs=pl.BlockSpec((_MM_BLK, _D), lambda i: (i, 0)),
        out_shape=jax.ShapeDtypeStruct((_N, _D), jnp.float32),
    )(x, w, b2d)


@functools.partial(
    pl.kernel,
    out_type=jax.ShapeDtypeStruct((_N, _D), jnp.float32),
    mesh=plsc.VectorSubcoreMesh(core_axis_name="c", subcore_axis_name="s"),
    scratch_types=[
        pltpu.VMEM_SHARED((_N, _D), jnp.float32),
        pltpu.VMEM((_H1, 1, _EC), jnp.int32),
        pltpu.VMEM((2, _H1, _EC), jnp.int32),
        pltpu.VMEM((2, _EC, _D), jnp.float32),
        pltpu.VMEM((_CHUNK_NODES, _D), jnp.float32),
        pltpu.VMEM((2, _CHUNK_NODES, _D), jnp.float32),
        pltpu.SemaphoreType.DMA,
        pltpu.SemaphoreType.DMA,
        pltpu.SemaphoreType.DMA,
    ],
)
def _sc_gin(h_hbm, afb_hbm, bfa_hbm, out_hbm,
            h_sh, afb_v, a2a_v, rows_v, hself_v, out_v,
            sem_idx, sem_rows, sem_out):
    c = lax.axis_index("c")
    s = lax.axis_index("s")
    wid = s * 2 + c
    cstart = wid * _CPW + jnp.minimum(wid, _REM)
    cn = _CPW + jnp.where(wid < _REM, 1, 0)
    start2 = cstart + cn - _H2

    def fire_a2a(h):
        def body(i, carry):
            pltpu.async_copy(bfa_hbm.at[afb_v.at[i, 0]], a2a_v.at[h, i],
                             sem_idx)
            return carry
        return body

    def drain_a2a(h):
        def body(i, carry):
            pltpu.make_async_copy(bfa_hbm.at[afb_v.at[i, 0]],
                                  a2a_v.at[h, i], sem_idx).wait()
            return carry
        return body

    pltpu.sync_copy(afb_hbm.at[pl.ds(cstart, _H1)], afb_v.at[pl.ds(0, _H1)])
    lax.fori_loop(0, _H1, fire_a2a(0), 0)

    @pl.when(s < 15)
    def _stage():
        pltpu.sync_copy(h_hbm.at[pl.ds(s * 624, 624)],
                        h_sh.at[pl.ds(s * 624, 624)])

    @pl.when(s == 15)
    def _stage_last():
        pltpu.sync_copy(h_hbm.at[pl.ds(9360, 640)],
                        h_sh.at[pl.ds(9360, 640)])

    lax.fori_loop(0, _H1, drain_a2a(0), 0)
    pltpu.sync_copy(afb_hbm.at[pl.ds(start2, _H2)], afb_v.at[pl.ds(0, _H2)])
    lax.fori_loop(0, _H2, fire_a2a(1), 0)

    plsc.subcore_barrier()

    def gather_rows(h, i, buf):
        pltpu.async_copy(h_sh.at[a2a_v.at[h, i]], rows_v.at[buf], sem_rows)

    def wait_rows(h, i, buf):
        pltpu.make_async_copy(h_sh.at[a2a_v.at[h, i]], rows_v.at[buf],
                              sem_rows).wait()

    def wait_out(nb, buf):
        pltpu.make_async_copy(out_v.at[buf],
                              out_hbm.at[pl.ds(nb, _CHUNK_NODES)],
                              sem_out).wait()

    def run_half(h, hstart, hcn):
        def chunk_body(i, carry):
            buf = lax.rem(i, 2)
            nb = (hstart + i) * _CHUNK_NODES
            wait_rows(h, i, buf)

            @pl.when(i + 1 < hcn)
            def _prefetch():
                gather_rows(h, i + 1, 1 - buf)

            @pl.when(i >= 2)
            def _drain_store():
                wait_out(nb - 2 * _CHUNK_NODES, buf)

            pltpu.sync_copy(h_sh.at[pl.ds(nb, _CHUNK_NODES)], hself_v)
            for j in range(_CHUNK_NODES):
                accs = tuple(hself_v[j, pl.ds(d * _LANES, _LANES)]
                             for d in range(_DV))

                def k_body(k, a, _j=j):
                    return tuple(
                        a[d] + rows_v[buf, _j * _DEG + k,
                                      pl.ds(d * _LANES, _LANES)]
                        for d in range(_DV))

                accs = lax.fori_loop(0, _DEG, k_body, accs, unroll=8)
                for d in range(_DV):
                    out_v[buf, j, pl.ds(d * _LANES, _LANES)] = jnp.maximum(
                        accs[d], 0.0)

            pltpu.async_copy(out_v.at[buf],
                             out_hbm.at[pl.ds(nb, _CHUNK_NODES)], sem_out)
            return carry

        gather_rows(h, 0, 0)
        lax.fori_loop(0, hcn, chunk_body, 0)
        wait_out((hstart + hcn - 2) * _CHUNK_NODES, lax.rem(hcn - 2, 2))
        wait_out((hstart + hcn - 1) * _CHUNK_NODES, lax.rem(hcn - 1, 2))


    run_half(0, cstart, _H1)
    lax.fori_loop(0, _H2, drain_a2a(1), 0)
    run_half(1, start2, _H2)


def kernel(x, b_from_a, a_from_b, W, b):
    afb = a_from_b.reshape(_CHUNKS, 1, _EC)
    h = _linear(x, W, b.reshape(1, _D))
    return _sc_gin(h, afb, b_from_a)
